# Initial kernel scaffold; baseline (speedup 1.0000x reference)
import jax
import jax.numpy as jnp
from jax.experimental import pallas as pl


def kernel(x, edge_index, edge_weight, W1, b1, W2, b2):
    raise NotImplementedError("write your pallas kernel here")


# speedup vs baseline: 9.6784x; 9.6784x over previous
import functools

import jax
import jax.numpy as jnp
from jax import lax
from jax.experimental import pallas as pl
from jax.experimental.pallas import tpu as pltpu
from jax.experimental.pallas import tpu_sc as plsc

N = 10000
E = 320000
D = 128

NC = 2
NS = 16
L = 16
NW = NC * NS

NP = 10240
NPR = NP // NS
E_EXT = 330240
EPW = E_EXT // NW
K = 80
NCHUNK = EPW // K


def _sc_mesh():
    return plsc.VectorSubcoreMesh(core_axis_name="c", subcore_axis_name="s")


@functools.partial(
    pl.kernel,
    out_type=jax.ShapeDtypeStruct((NC, NP), jnp.float32),
    mesh=_sc_mesh(),
    scratch_types=[
        pltpu.VMEM((K,), jnp.int32),
        pltpu.VMEM((K,), jnp.float32),
        pltpu.VMEM_SHARED((NP,), jnp.float32),
    ],
)
def _deg_sc(col_hbm, ew_hbm, zeros_hbm, out_hbm, cidx_v, ew_v, acc_sh):
    c = lax.axis_index("c")
    s = lax.axis_index("s")
    wid = s * NC + c
    pltpu.sync_copy(zeros_hbm.at[pl.ds(s * NPR, NPR)], acc_sh.at[pl.ds(s * NPR, NPR)])
    plsc.subcore_barrier()

    base = wid * EPW

    def chunk(kk, carry):
        off = base + kk * K
        pltpu.sync_copy(col_hbm.at[pl.ds(off, K)], cidx_v)
        pltpu.sync_copy(ew_hbm.at[pl.ds(off, K)], ew_v)
        pltpu.sync_copy(ew_v, acc_sh.at[cidx_v], add=True)
        return carry

    lax.fori_loop(0, NCHUNK, chunk, jnp.int32(0))
    plsc.subcore_barrier()
    pltpu.sync_copy(acc_sh.at[pl.ds(s * NPR, NPR)], out_hbm.at[c, pl.ds(s * NPR, NPR)])


@functools.partial(
    pl.kernel,
    out_type=jax.ShapeDtypeStruct((NC, NP, D), jnp.float32),
    mesh=_sc_mesh(),
    scratch_types=[
        pltpu.VMEM((K,), jnp.int32),
        pltpu.VMEM((K,), jnp.int32),
        pltpu.VMEM((K,), jnp.float32),
        pltpu.VMEM((K, D), jnp.float32),
        pltpu.VMEM((NP,), jnp.float32),
        pltpu.VMEM_SHARED((NP, D), jnp.float32),
        pltpu.SemaphoreType.DMA,
    ],
    compiler_params=pltpu.CompilerParams(needs_layout_passes=False),
)
def _prop_sc(row_hbm, col_hbm, ew_hbm, dis_hbm, xw_hbm, zeros_hbm, out_hbm,
             ridx_v, cidx_v, ew_v, rows_v, dis_v, acc_sh, sem):
    c = lax.axis_index("c")
    s = lax.axis_index("s")
    wid = s * NC + c
    pltpu.sync_copy(dis_hbm, dis_v)
    pltpu.sync_copy(zeros_hbm.at[pl.ds(s * NPR, NPR), :],
                    acc_sh.at[pl.ds(s * NPR, NPR), :])
    plsc.subcore_barrier()

    base = wid * EPW

    def chunk(kk, carry):
        off = base + kk * K
        pltpu.sync_copy(row_hbm.at[pl.ds(off, K)], ridx_v)
        pltpu.sync_copy(col_hbm.at[pl.ds(off, K)], cidx_v)
        pltpu.sync_copy(ew_hbm.at[pl.ds(off, K)], ew_v)
        pltpu.async_copy(xw_hbm.at[ridx_v], rows_v, sem).wait()
        def scale_group(g, cy):
            sl = pl.ds(g * L, L)
            wv = (plsc.load_gather(dis_v, [ridx_v[sl]]) * ew_v[sl]
                  * plsc.load_gather(dis_v, [cidx_v[sl]]))
            ebase = g * L
            for j in range(L):
                bv = jnp.full((L,), wv[j], dtype=jnp.float32)
                for d in range(D // L):
                    fsl = pl.ds(d * L, L)
                    rows_v[ebase + j, fsl] = rows_v[ebase + j, fsl] * bv
            return cy

        lax.fori_loop(0, K // L, scale_group, carry)
        pltpu.sync_copy(rows_v, acc_sh.at[cidx_v], add=True)
        return carry

    lax.fori_loop(0, NCHUNK, chunk, jnp.int32(0))
    plsc.subcore_barrier()
    pltpu.sync_copy(acc_sh.at[pl.ds(s * NPR, NPR), :],
                    out_hbm.at[c, pl.ds(s * NPR, NPR), :])


def _tc1_body(degp_ref, x_ref, w_ref, dis_ref, xw_ref):
    deg = degp_ref[0] + degp_ref[1]
    dis_ref[...] = jnp.where(deg > 0.0, lax.rsqrt(deg), 0.0)
    xw_ref[...] = jnp.dot(x_ref[...], w_ref[...],
                          preferred_element_type=jnp.float32)


def _tc2_body(p_ref, b_ref, w_ref, xw_ref):
    h = jnp.maximum(p_ref[0, :N, :] + p_ref[1, :N, :] + b_ref[...], 0.0)
    xw_ref[...] = jnp.dot(h, w_ref[...], preferred_element_type=jnp.float32)


def _tc3_body(p_ref, b_ref, out_ref):
    out_ref[...] = p_ref[0, :N, :] + p_ref[1, :N, :] + b_ref[...]


_tc1 = pl.pallas_call(
    _tc1_body,
    out_shape=(jax.ShapeDtypeStruct((NP // D, D), jnp.float32),
               jax.ShapeDtypeStruct((N, D), jnp.float32)),
)
_tc2 = pl.pallas_call(
    _tc2_body,
    out_shape=jax.ShapeDtypeStruct((N, D), jnp.float32),
)
_tc3 = pl.pallas_call(
    _tc3_body,
    out_shape=jax.ShapeDtypeStruct((N, D), jnp.float32),
)


def kernel(x, edge_index, edge_weight, W1, b1, W2, b2):
    row = edge_index[0].astype(jnp.int32)
    col = edge_index[1].astype(jnp.int32)
    ew = edge_weight.astype(jnp.float32)

    loop_idx = jnp.arange(N, dtype=jnp.int32)
    npad = E_EXT - E - N
    zpad_i = jnp.zeros((npad,), jnp.int32)
    row_e = jnp.concatenate([row, loop_idx, zpad_i])
    col_e = jnp.concatenate([col, loop_idx, zpad_i])
    ew_e = jnp.concatenate([ew, jnp.ones((N,), jnp.float32),
                            jnp.zeros((npad,), jnp.float32)])

    zeros1 = jnp.zeros((NP,), jnp.float32)
    zerosR = jnp.zeros((NP, D), jnp.float32)

    degp = _deg_sc(col_e, ew_e, zeros1)
    dis80, xw1 = _tc1(degp.reshape(NC, NP // D, D), x, W1)
    disf = dis80.reshape(NP)
    p1 = _prop_sc(row_e, col_e, ew_e, disf, xw1, zerosR)
    xw2 = _tc2(p1, b1.reshape(1, D), W2)
    p2 = _prop_sc(row_e, col_e, ew_e, disf, xw2, zerosR)
    out = _tc3(p2, b2.reshape(1, D))
    return out


# speedup vs baseline: 18.7797x; 1.9404x over previous
import functools

import jax
import jax.numpy as jnp
from jax import lax
from jax.experimental import pallas as pl
from jax.experimental.pallas import tpu as pltpu
from jax.experimental.pallas import tpu_sc as plsc

N = 10000
E = 320000
D = 128

NC = 2
NS = 16
L = 16
NW = NC * NS

NP = 10240
NPR = NP // NS
K = 64
NCHUNK = 162
EPW = NCHUNK * K
E_EXT = NW * EPW
NBUF = 2
DEG_W = 8


def _sc_mesh():
    return plsc.VectorSubcoreMesh(core_axis_name="c", subcore_axis_name="s")


@functools.partial(
    pl.kernel,
    out_type=jax.ShapeDtypeStruct((NC, NP), jnp.float32),
    mesh=_sc_mesh(),
    scratch_types=[
        pltpu.VMEM((EPW,), jnp.int32),
        pltpu.VMEM((EPW,), jnp.float32),
        pltpu.VMEM_SHARED((NP,), jnp.float32),
        pltpu.SemaphoreType.DMA,
    ],
    compiler_params=pltpu.CompilerParams(needs_layout_passes=False),
)
def _deg_sc(col3_hbm, ew3_hbm, zeros_hbm, out_hbm, cidx_all, ew_all, acc_sh, dsem):
    c = lax.axis_index("c")
    s = lax.axis_index("s")
    wid = s * NC + c
    pltpu.sync_copy(col3_hbm.at[wid], cidx_all)
    pltpu.sync_copy(ew3_hbm.at[wid], ew_all)
    pltpu.sync_copy(zeros_hbm.at[pl.ds(s * NPR, NPR)], acc_sh.at[pl.ds(s * NPR, NPR)])
    plsc.subcore_barrier()

    def chunk(kk, carry):
        pltpu.async_copy(ew_all.at[pl.ds(kk * K, K)],
                         acc_sh.at[cidx_all.at[pl.ds(kk * K, K)]], dsem, add=True)

        @pl.when(kk >= DEG_W)
        def _():
            kp = kk - DEG_W
            pltpu.make_async_copy(ew_all.at[pl.ds(kp * K, K)],
                                  acc_sh.at[cidx_all.at[pl.ds(kp * K, K)]],
                                  dsem).wait()

        return carry

    lax.fori_loop(0, NCHUNK, chunk, jnp.int32(0))

    def drain(w, carry):
        kk = NCHUNK - DEG_W + w
        pltpu.make_async_copy(ew_all.at[pl.ds(kk * K, K)],
                              acc_sh.at[cidx_all.at[pl.ds(kk * K, K)]],
                              dsem).wait()
        return carry

    lax.fori_loop(0, DEG_W, drain, jnp.int32(0))
    plsc.subcore_barrier()
    pltpu.sync_copy(acc_sh.at[pl.ds(s * NPR, NPR)], out_hbm.at[c, pl.ds(s * NPR, NPR)])


@functools.partial(
    pl.kernel,
    out_type=jax.ShapeDtypeStruct((NC, NP, D), jnp.float32),
    mesh=_sc_mesh(),
    scratch_types=[
        pltpu.VMEM((EPW,), jnp.int32),
        pltpu.VMEM((EPW,), jnp.int32),
        pltpu.VMEM((EPW,), jnp.float32),
        pltpu.VMEM((K, D), jnp.float32),
        pltpu.VMEM((K, D), jnp.float32),
        pltpu.VMEM_SHARED((NP, D), jnp.float32),
        pltpu.SemaphoreType.DMA,
        pltpu.SemaphoreType.DMA,
        pltpu.SemaphoreType.DMA,
        pltpu.SemaphoreType.DMA,
    ],
    compiler_params=pltpu.CompilerParams(needs_layout_passes=False),
)
def _prop_sc(row3_hbm, col3_hbm, ew3_hbm, xw_hbm, zeros_hbm, out_hbm,
             ridx_all, cidx_all, ew_all, rows0, rows1, acc_sh,
             g0, g1, s0, s1):
    c = lax.axis_index("c")
    s = lax.axis_index("s")
    wid = s * NC + c
    rows = (rows0, rows1)
    gsem = (g0, g1)
    ssem = (s0, s1)

    pltpu.sync_copy(row3_hbm.at[wid], ridx_all)
    pltpu.sync_copy(col3_hbm.at[wid], cidx_all)
    pltpu.sync_copy(ew3_hbm.at[wid], ew_all)
    pltpu.sync_copy(zeros_hbm.at[pl.ds(s * NPR, NPR), :],
                    acc_sh.at[pl.ds(s * NPR, NPR), :])
    plsc.subcore_barrier()

    pltpu.async_copy(xw_hbm.at[ridx_all.at[pl.ds(0, K)]], rows[0], gsem[0])

    def step(t, carry):
        for b in range(NBUF):
            kk = NBUF * t + b
            rb, gb, sb = rows[b], gsem[b], ssem[b]
            bo = 1 - b

            @pl.when(kk >= 1)
            def _():
                pltpu.make_async_copy(rows[bo],
                                      acc_sh.at[cidx_all.at[pl.ds((kk - 1) * K, K)]],
                                      ssem[bo]).wait()

            @pl.when(kk + 1 < NCHUNK)
            def _():
                pltpu.async_copy(xw_hbm.at[ridx_all.at[pl.ds((kk + 1) * K, K)]],
                                 rows[bo], gsem[bo])

            pltpu.make_async_copy(xw_hbm.at[ridx_all.at[pl.ds(kk * K, K)]],
                                  rb, gb).wait()

            def scale_group(g, cy):
                wv = ew_all[pl.ds(kk * K + g * L, L)]
                ebase = g * L
                for j in range(L):
                    bv = jnp.full((L,), wv[j], dtype=jnp.float32)
                    for d in range(D // L):
                        fsl = pl.ds(d * L, L)
                        rb[ebase + j, fsl] = rb[ebase + j, fsl] * bv
                return cy

            lax.fori_loop(0, K // L, scale_group, carry)

            pltpu.async_copy(rb, acc_sh.at[cidx_all.at[pl.ds(kk * K, K)]],
                             sb, add=True)

        return carry

    lax.fori_loop(0, NCHUNK // NBUF, step, jnp.int32(0))
    pltpu.make_async_copy(rows[(NCHUNK - 1) % NBUF],
                          acc_sh.at[cidx_all.at[pl.ds((NCHUNK - 1) * K, K)]],
                          ssem[(NCHUNK - 1) % NBUF]).wait()
    plsc.subcore_barrier()
    pltpu.sync_copy(acc_sh.at[pl.ds(s * NPR, NPR), :],
                    out_hbm.at[c, pl.ds(s * NPR, NPR), :])


def _tc1_body(degp_ref, x_ref, w_ref, dis_ref, y_ref):
    deg = degp_ref[0] + degp_ref[1]
    dis = jnp.where(deg > 0.0, lax.rsqrt(deg), 0.0)
    dis_ref[...] = dis
    y_ref[...] = jnp.dot(x_ref[...] * dis[:N], w_ref[...],
                         preferred_element_type=jnp.float32)


def _tc2_body(p_ref, dis_ref, b_ref, w_ref, y_ref):
    disn = dis_ref[:N]
    h = jnp.maximum(disn * (p_ref[0, :N, :] + p_ref[1, :N, :]) + b_ref[...],
                    0.0)
    y_ref[...] = jnp.dot(disn * h, w_ref[...],
                         preferred_element_type=jnp.float32)


def _tc3_body(p_ref, dis_ref, b_ref, out_ref):
    out_ref[...] = (dis_ref[:N] * (p_ref[0, :N, :] + p_ref[1, :N, :])
                    + b_ref[...])


_tc1 = pl.pallas_call(
    _tc1_body,
    out_shape=(jax.ShapeDtypeStruct((NP, 1), jnp.float32),
               jax.ShapeDtypeStruct((N, D), jnp.float32)),
)
_tc2 = pl.pallas_call(
    _tc2_body,
    out_shape=jax.ShapeDtypeStruct((N, D), jnp.float32),
)
_tc3 = pl.pallas_call(
    _tc3_body,
    out_shape=jax.ShapeDtypeStruct((N, D), jnp.float32),
)


def kernel(x, edge_index, edge_weight, W1, b1, W2, b2):
    row = edge_index[0].astype(jnp.int32)
    col = edge_index[1].astype(jnp.int32)
    ew = edge_weight.astype(jnp.float32)

    loop_idx = jnp.arange(N, dtype=jnp.int32)
    npad = E_EXT - E - N
    zpad_i = jnp.zeros((npad,), jnp.int32)
    row_e = jnp.concatenate([row, loop_idx, zpad_i]).reshape(NW, EPW)
    col_e = jnp.concatenate([col, loop_idx, zpad_i]).reshape(NW, EPW)
    ew_e = jnp.concatenate([ew, jnp.ones((N,), jnp.float32),
                            jnp.zeros((npad,), jnp.float32)]).reshape(NW, EPW)

    zeros1 = jnp.zeros((NP,), jnp.float32)
    zerosR = jnp.zeros((NP, D), jnp.float32)

    degp = _deg_sc(col_e, ew_e, zeros1)
    dis, y1 = _tc1(degp.reshape(NC, NP, 1), x, W1)
    p1 = _prop_sc(row_e, col_e, ew_e, y1, zerosR)
    y2 = _tc2(p1, dis, b1.reshape(1, D), W2)
    p2 = _prop_sc(row_e, col_e, ew_e, y2, zerosR)
    out = _tc3(p2, dis, b2.reshape(1, D))
    return out


# speedup vs baseline: 19.2682x; 1.0260x over previous
import functools

import jax
import jax.numpy as jnp
from jax import lax
from jax.experimental import pallas as pl
from jax.experimental.pallas import tpu as pltpu
from jax.experimental.pallas import tpu_sc as plsc

N = 10000
E = 320000
D = 128

NC = 2
NS = 16
L = 16
NW = NC * NS

NP = 10240
NPR = NP // NS
K = 32
NCHUNK = 324
EPW = NCHUNK * K
E_EXT = NW * EPW
NBUF = 4
DEG_W = 8


def _sc_mesh():
    return plsc.VectorSubcoreMesh(core_axis_name="c", subcore_axis_name="s")


@functools.partial(
    pl.kernel,
    out_type=jax.ShapeDtypeStruct((NC, NP), jnp.float32),
    mesh=_sc_mesh(),
    scratch_types=[
        pltpu.VMEM((EPW,), jnp.int32),
        pltpu.VMEM((EPW,), jnp.float32),
        pltpu.VMEM_SHARED((NP,), jnp.float32),
        pltpu.SemaphoreType.DMA,
    ],
    compiler_params=pltpu.CompilerParams(needs_layout_passes=False),
)
def _deg_sc(col3_hbm, ew3_hbm, zeros_hbm, out_hbm, cidx_all, ew_all, acc_sh, dsem):
    c = lax.axis_index("c")
    s = lax.axis_index("s")
    wid = s * NC + c
    pltpu.sync_copy(col3_hbm.at[wid], cidx_all)
    pltpu.sync_copy(ew3_hbm.at[wid], ew_all)
    pltpu.sync_copy(zeros_hbm.at[pl.ds(s * NPR, NPR)], acc_sh.at[pl.ds(s * NPR, NPR)])
    plsc.subcore_barrier()

    def chunk(kk, carry):
        pltpu.async_copy(ew_all.at[pl.ds(kk * K, K)],
                         acc_sh.at[cidx_all.at[pl.ds(kk * K, K)]], dsem, add=True)

        @pl.when(kk >= DEG_W)
        def _():
            kp = kk - DEG_W
            pltpu.make_async_copy(ew_all.at[pl.ds(kp * K, K)],
                                  acc_sh.at[cidx_all.at[pl.ds(kp * K, K)]],
                                  dsem).wait()

        return carry

    lax.fori_loop(0, NCHUNK, chunk, jnp.int32(0))

    def drain(w, carry):
        kk = NCHUNK - DEG_W + w
        pltpu.make_async_copy(ew_all.at[pl.ds(kk * K, K)],
                              acc_sh.at[cidx_all.at[pl.ds(kk * K, K)]],
                              dsem).wait()
        return carry

    lax.fori_loop(0, DEG_W, drain, jnp.int32(0))
    plsc.subcore_barrier()
    pltpu.sync_copy(acc_sh.at[pl.ds(s * NPR, NPR)], out_hbm.at[c, pl.ds(s * NPR, NPR)])


@functools.partial(
    pl.kernel,
    out_type=jax.ShapeDtypeStruct((NC, NP, D), jnp.float32),
    mesh=_sc_mesh(),
    scratch_types=[
        pltpu.VMEM((EPW,), jnp.int32),
        pltpu.VMEM((EPW,), jnp.int32),
        pltpu.VMEM((EPW,), jnp.float32),
        pltpu.VMEM((K, D), jnp.float32),
        pltpu.VMEM((K, D), jnp.float32),
        pltpu.VMEM((K, D), jnp.float32),
        pltpu.VMEM((K, D), jnp.float32),
        pltpu.VMEM_SHARED((NP, D), jnp.float32),
        pltpu.SemaphoreType.DMA,
        pltpu.SemaphoreType.DMA,
        pltpu.SemaphoreType.DMA,
        pltpu.SemaphoreType.DMA,
        pltpu.SemaphoreType.DMA,
        pltpu.SemaphoreType.DMA,
        pltpu.SemaphoreType.DMA,
        pltpu.SemaphoreType.DMA,
    ],
    compiler_params=pltpu.CompilerParams(needs_layout_passes=False),
)
def _prop_sc(row3_hbm, col3_hbm, ew3_hbm, xw_hbm, zeros_hbm, out_hbm,
             ridx_all, cidx_all, ew_all, rows0, rows1, rows2, rows3, acc_sh,
             g0, g1, g2, g3, s0, s1, s2, s3):
    c = lax.axis_index("c")
    s = lax.axis_index("s")
    wid = s * NC + c
    rows = (rows0, rows1, rows2, rows3)
    gsem = (g0, g1, g2, g3)
    ssem = (s0, s1, s2, s3)

    pltpu.sync_copy(row3_hbm.at[wid], ridx_all)
    pltpu.sync_copy(col3_hbm.at[wid], cidx_all)
    pltpu.sync_copy(ew3_hbm.at[wid], ew_all)
    pltpu.sync_copy(zeros_hbm.at[pl.ds(s * NPR, NPR), :],
                    acc_sh.at[pl.ds(s * NPR, NPR), :])
    plsc.subcore_barrier()

    for i in range(NBUF - 1):
        pltpu.async_copy(xw_hbm.at[ridx_all.at[pl.ds(i * K, K)]],
                         rows[i], gsem[i])

    def step(t, carry):
        for b in range(NBUF):
            kk = NBUF * t + b
            rb, gb, sb = rows[b], gsem[b], ssem[b]
            bo = (b + NBUF - 1) % NBUF

            @pl.when(kk >= 1)
            def _():
                pltpu.make_async_copy(rows[bo],
                                      acc_sh.at[cidx_all.at[pl.ds((kk - 1) * K, K)]],
                                      ssem[bo]).wait()

            @pl.when(kk + NBUF - 1 < NCHUNK)
            def _():
                pltpu.async_copy(
                    xw_hbm.at[ridx_all.at[pl.ds((kk + NBUF - 1) * K, K)]],
                    rows[bo], gsem[bo])

            pltpu.make_async_copy(xw_hbm.at[ridx_all.at[pl.ds(kk * K, K)]],
                                  rb, gb).wait()

            def scale_group(g, cy):
                wv = ew_all[pl.ds(kk * K + g * L, L)]
                ebase = g * L
                for j in range(L):
                    bv = jnp.full((L,), wv[j], dtype=jnp.float32)
                    for d in range(D // L):
                        fsl = pl.ds(d * L, L)
                        rb[ebase + j, fsl] = rb[ebase + j, fsl] * bv
                return cy

            lax.fori_loop(0, K // L, scale_group, carry)

            pltpu.async_copy(rb, acc_sh.at[cidx_all.at[pl.ds(kk * K, K)]],
                             sb, add=True)

        return carry

    lax.fori_loop(0, NCHUNK // NBUF, step, jnp.int32(0))
    pltpu.make_async_copy(rows[(NCHUNK - 1) % NBUF],
                          acc_sh.at[cidx_all.at[pl.ds((NCHUNK - 1) * K, K)]],
                          ssem[(NCHUNK - 1) % NBUF]).wait()
    plsc.subcore_barrier()
    pltpu.sync_copy(acc_sh.at[pl.ds(s * NPR, NPR), :],
                    out_hbm.at[c, pl.ds(s * NPR, NPR), :])


def _tc1_body(degp_ref, x_ref, w_ref, dis_ref, y_ref):
    deg = degp_ref[0] + degp_ref[1]
    dis = jnp.where(deg > 0.0, lax.rsqrt(deg), 0.0)
    dis_ref[...] = dis
    y_ref[...] = jnp.dot(x_ref[...] * dis[:N], w_ref[...],
                         preferred_element_type=jnp.float32)


def _tc2_body(p_ref, dis_ref, b_ref, w_ref, y_ref):
    disn = dis_ref[:N]
    h = jnp.maximum(disn * (p_ref[0, :N, :] + p_ref[1, :N, :]) + b_ref[...],
                    0.0)
    y_ref[...] = jnp.dot(disn * h, w_ref[...],
                         preferred_element_type=jnp.float32)


def _tc3_body(p_ref, dis_ref, b_ref, out_ref):
    out_ref[...] = (dis_ref[:N] * (p_ref[0, :N, :] + p_ref[1, :N, :])
                    + b_ref[...])


_tc1 = pl.pallas_call(
    _tc1_body,
    out_shape=(jax.ShapeDtypeStruct((NP, 1), jnp.float32),
               jax.ShapeDtypeStruct((N, D), jnp.float32)),
)
_tc2 = pl.pallas_call(
    _tc2_body,
    out_shape=jax.ShapeDtypeStruct((N, D), jnp.float32),
)
_tc3 = pl.pallas_call(
    _tc3_body,
    out_shape=jax.ShapeDtypeStruct((N, D), jnp.float32),
)


def kernel(x, edge_index, edge_weight, W1, b1, W2, b2):
    row = edge_index[0].astype(jnp.int32)
    col = edge_index[1].astype(jnp.int32)
    ew = edge_weight.astype(jnp.float32)

    loop_idx = jnp.arange(N, dtype=jnp.int32)
    npad = E_EXT - E - N
    zpad_i = jnp.zeros((npad,), jnp.int32)
    row_e = jnp.concatenate([row, loop_idx, zpad_i]).reshape(NW, EPW)
    col_e = jnp.concatenate([col, loop_idx, zpad_i]).reshape(NW, EPW)
    ew_e = jnp.concatenate([ew, jnp.ones((N,), jnp.float32),
                            jnp.zeros((npad,), jnp.float32)]).reshape(NW, EPW)

    zeros1 = jnp.zeros((NP,), jnp.float32)
    zerosR = jnp.zeros((NP, D), jnp.float32)

    degp = _deg_sc(col_e, ew_e, zeros1)
    dis, y1 = _tc1(degp.reshape(NC, NP, 1), x, W1)
    p1 = _prop_sc(row_e, col_e, ew_e, y1, zerosR)
    y2 = _tc2(p1, dis, b1.reshape(1, D), W2)
    p2 = _prop_sc(row_e, col_e, ew_e, y2, zerosR)
    out = _tc3(p2, dis, b2.reshape(1, D))
    return out
